# Initial kernel scaffold; baseline (speedup 1.0000x reference)
import jax
import jax.numpy as jnp
from jax.experimental import pallas as pl


def kernel(edge_index, xc, xt, W, b):
    raise NotImplementedError("write your pallas kernel here")


# speedup vs baseline: 3.9319x; 3.9319x over previous
import functools

import jax
import jax.numpy as jnp
from jax import lax
from jax.experimental import pallas as pl
from jax.experimental.pallas import tpu as pltpu
from jax.experimental.pallas import tpu_sc as plsc

_N = 10000
_E = 320000
_H = 128
_C = 5

_NC = 2
_NS = 16
_NW = _NC * _NS
_EPW = _E // _NW
_K = 80
_NCHUNK = _EPW // _K
_G = _K // 16

_RBLK = 1000


def _prep_body(x_ref, w_ref, o_ref):
    comp = pl.program_id(0) // (_N // _RBLK)
    o_ref[...] = x_ref[...] * w_ref[pl.ds(comp, 1), :]


@jax.jit
def _fold_weights(xcf, w2):
    return pl.pallas_call(
        _prep_body,
        grid=(_C * _N // _RBLK,),
        in_specs=[
            pl.BlockSpec((_RBLK, _H), lambda p: (p, 0)),
            pl.BlockSpec((_C, _H), lambda p: (0, 0)),
        ],
        out_specs=pl.BlockSpec((_RBLK, _H), lambda p: (p, 0)),
        out_shape=jax.ShapeDtypeStruct((_C * _N, _H), jnp.float32),
    )(xcf, w2)


def _edge_body(src_hbm, dst_hbm, xc_hbm, xt_hbm, bias_hbm, out_hbm,
               src_v, dst_v, sidx_v, didx_v, a_v, b_v, t_v, bias_v, out_v,
               sem):
    wid = lax.axis_index("s") * _NC + lax.axis_index("c")
    base_w = wid * _EPW

    pltpu.sync_copy(bias_hbm, bias_v)
    bsum = bias_v[...]

    def chunk_body(c, carry):
        base = base_w + c * _K
        pltpu.sync_copy(src_hbm.at[pl.ds(base, _K)], src_v)
        pltpu.sync_copy(dst_hbm.at[pl.ds(base, _K)], dst_v)
        for i in range(_C):
            off = i * _N
            for j in range(_G):
                sl = pl.ds(j * 16, 16)
                sidx_v[sl] = src_v[sl] + off
                didx_v[sl] = dst_v[sl] + off
            cp_a = pltpu.async_copy(xc_hbm.at[sidx_v], a_v, sem)
            cp_b = pltpu.async_copy(xt_hbm.at[didx_v], b_v, sem)
            cp_a.wait()
            cp_b.wait()

            lanes = lax.iota(jnp.int32, 16) * 16
            for g in range(_G):
                def e_body(t, carry):
                    e = g * 16 + t
                    acc0 = a_v[e, pl.ds(0, 16)] * b_v[e, pl.ds(0, 16)]
                    acc1 = a_v[e, pl.ds(16, 16)] * b_v[e, pl.ds(16, 16)]
                    for hb in range(2, _H // 16, 2):
                        sl0 = pl.ds(hb * 16, 16)
                        sl1 = pl.ds(hb * 16 + 16, 16)
                        acc0 = acc0 + a_v[e, sl0] * b_v[e, sl0]
                        acc1 = acc1 + a_v[e, sl1] * b_v[e, sl1]
                    t_v[pl.ds(t * 16, 16)] = acc0 + acc1
                    return carry

                lax.fori_loop(0, 16, e_body, 0, unroll=4)
                s = plsc.load_gather(t_v, [lanes])
                for l in range(1, 16):
                    s = s + plsc.load_gather(t_v, [lanes + l])
                sl = pl.ds(g * 16, 16)
                if i == 0:
                    out_v[sl] = s + bsum
                else:
                    out_v[sl] = out_v[sl] + s
        pltpu.sync_copy(out_v, out_hbm.at[pl.ds(base, _K)])
        return carry

    lax.fori_loop(0, _NCHUNK, chunk_body, 0)


@jax.jit
def _edge_scores(src, dst, xcw, xtf, b16):
    mesh = plsc.VectorSubcoreMesh(core_axis_name="c", subcore_axis_name="s")
    fn = functools.partial(
        pl.kernel,
        out_type=jax.ShapeDtypeStruct((_E,), jnp.float32),
        mesh=mesh,
        compiler_params=pltpu.CompilerParams(needs_layout_passes=False),
        scratch_types=[
            pltpu.VMEM((_K,), jnp.int32),
            pltpu.VMEM((_K,), jnp.int32),
            pltpu.VMEM((_K,), jnp.int32),
            pltpu.VMEM((_K,), jnp.int32),
            pltpu.VMEM((_K, _H), jnp.float32),
            pltpu.VMEM((_K, _H), jnp.float32),
            pltpu.VMEM((256,), jnp.float32),
            pltpu.VMEM((16,), jnp.float32),
            pltpu.VMEM((_K,), jnp.float32),
            pltpu.SemaphoreType.DMA,
        ],
    )(_edge_body)
    return fn(src, dst, xcw, xtf, b16)


def kernel(edge_index, xc, xt, W, b):
    src = edge_index[0].astype(jnp.int32)
    dst = edge_index[1].astype(jnp.int32)
    xcf = xc.reshape(_C * _N, _H)
    xtf = xt.reshape(_C * _N, _H)
    w2 = W.reshape(_C, _H)
    xcw = _fold_weights(xcf, w2)
    b16 = jnp.full((16,), jnp.sum(b), jnp.float32)
    out = _edge_scores(src, dst, xcw, xtf, b16)
    return out.reshape(_E, 1)


# speedup vs baseline: 6.1191x; 1.5562x over previous
import functools

import jax
import jax.numpy as jnp
from jax import lax
from jax.experimental import pallas as pl
from jax.experimental.pallas import tpu as pltpu
from jax.experimental.pallas import tpu_sc as plsc

_N = 10000
_E = 320000
_H = 128
_C = 5

_NC = 2
_NS = 16
_NW = _NC * _NS
_EPW = _E // _NW
_K = 80
_NCHUNK = _EPW // _K
_G = _K // 16

_RBLK = 1000


def _prep_body(x_ref, w_ref, o_ref):
    comp = pl.program_id(0) // (_N // _RBLK)
    o_ref[...] = x_ref[...] * w_ref[pl.ds(comp, 1), :]


@jax.jit
def _fold_weights(xcf, w2):
    return pl.pallas_call(
        _prep_body,
        grid=(_C * _N // _RBLK,),
        in_specs=[
            pl.BlockSpec((_RBLK, _H), lambda p: (p, 0)),
            pl.BlockSpec((_C, _H), lambda p: (0, 0)),
        ],
        out_specs=pl.BlockSpec((_RBLK, _H), lambda p: (p, 0)),
        out_shape=jax.ShapeDtypeStruct((_C * _N, _H), jnp.float32),
    )(xcf, w2)


def _edge_body(src_hbm, dst_hbm, xc_hbm, xt_hbm, bias_hbm, out_hbm,
               srcN_v, dstN_v, sidx_v, didx_v, a_v, b_v, t_v, bias_v, out_v,
               sems):
    wid = lax.axis_index("s") * _NC + lax.axis_index("c")
    base_w = wid * _EPW

    pltpu.sync_copy(bias_hbm, bias_v)
    bsum = bias_v[...]

    def load_next(c):
        base = base_w + c * _K
        pltpu.sync_copy(src_hbm.at[pl.ds(base, _K)], srcN_v)
        pltpu.sync_copy(dst_hbm.at[pl.ds(base, _K)], dstN_v)

    def fire(i):
        off = i * _N
        for j in range(_G):
            sl = pl.ds(j * 16, 16)
            sidx_v[i, sl] = srcN_v[sl] + off
            didx_v[i, sl] = dstN_v[sl] + off
        pltpu.async_copy(xc_hbm.at[sidx_v.at[i]], a_v.at[i], sems.at[i])
        pltpu.async_copy(xt_hbm.at[didx_v.at[i]], b_v.at[i], sems.at[i])

    def wait(i):
        pltpu.make_async_copy(xc_hbm.at[sidx_v.at[i]], a_v.at[i],
                              sems.at[i]).wait()
        pltpu.make_async_copy(xt_hbm.at[didx_v.at[i]], b_v.at[i],
                              sems.at[i]).wait()

    lanes = lax.iota(jnp.int32, 16) * 16

    def compute(i):
        for g in range(_G):
            def e_body(t, carry):
                e = g * 16 + t
                acc0 = a_v[i, e, pl.ds(0, 16)] * b_v[i, e, pl.ds(0, 16)]
                acc1 = a_v[i, e, pl.ds(16, 16)] * b_v[i, e, pl.ds(16, 16)]
                for hb in range(2, _H // 16, 2):
                    sl0 = pl.ds(hb * 16, 16)
                    sl1 = pl.ds(hb * 16 + 16, 16)
                    acc0 = acc0 + a_v[i, e, sl0] * b_v[i, e, sl0]
                    acc1 = acc1 + a_v[i, e, sl1] * b_v[i, e, sl1]
                t_v[pl.ds(t * 16, 16)] = acc0 + acc1
                return carry

            lax.fori_loop(0, 16, e_body, 0, unroll=4)
            s = plsc.load_gather(t_v, [lanes])
            for l in range(1, 16):
                s = s + plsc.load_gather(t_v, [lanes + l])
            sl = pl.ds(g * 16, 16)
            if i == 0:
                out_v[sl] = s + bsum
            else:
                out_v[sl] = out_v[sl] + s

    load_next(0)
    for i in range(_C):
        fire(i)

    def chunk_body(c, carry):
        @pl.when(c < _NCHUNK - 1)
        def _():
            load_next(c + 1)

        for i in range(_C):
            wait(i)
            compute(i)

            @pl.when(c < _NCHUNK - 1)
            def _():
                fire(i)

        pltpu.sync_copy(out_v, out_hbm.at[pl.ds(base_w + c * _K, _K)])
        return carry

    lax.fori_loop(0, _NCHUNK, chunk_body, 0)


@jax.jit
def _edge_scores(src, dst, xcw, xtf, b16):
    mesh = plsc.VectorSubcoreMesh(core_axis_name="c", subcore_axis_name="s")
    fn = functools.partial(
        pl.kernel,
        out_type=jax.ShapeDtypeStruct((_E,), jnp.float32),
        mesh=mesh,
        compiler_params=pltpu.CompilerParams(needs_layout_passes=False),
        scratch_types=[
            pltpu.VMEM((_K,), jnp.int32),
            pltpu.VMEM((_K,), jnp.int32),
            pltpu.VMEM((_C, _K), jnp.int32),
            pltpu.VMEM((_C, _K), jnp.int32),
            pltpu.VMEM((_C, _K, _H), jnp.float32),
            pltpu.VMEM((_C, _K, _H), jnp.float32),
            pltpu.VMEM((256,), jnp.float32),
            pltpu.VMEM((16,), jnp.float32),
            pltpu.VMEM((_K,), jnp.float32),
            pltpu.SemaphoreType.DMA((_C,)),
        ],
    )(_edge_body)
    return fn(src, dst, xcw, xtf, b16)


def kernel(edge_index, xc, xt, W, b):
    src = edge_index[0].astype(jnp.int32)
    dst = edge_index[1].astype(jnp.int32)
    xcf = xc.reshape(_C * _N, _H)
    xtf = xt.reshape(_C * _N, _H)
    w2 = W.reshape(_C, _H)
    xcw = _fold_weights(xcf, w2)
    b16 = jnp.full((16,), jnp.sum(b), jnp.float32)
    out = _edge_scores(src, dst, xcw, xtf, b16)
    return out.reshape(_E, 1)
